# baseline (speedup 1.0000x reference)
import dataclasses
import functools

import jax
import jax.numpy as jnp
from jax import lax
from jax.experimental import pallas as pl
from jax.experimental.pallas import tpu as pltpu
from jax.experimental.pallas import tpu_sc as plsc

N = 10000
E = 320000
D_IN = 128
D_HID = 128
D_OUT = 64
D_PE = 128

NC = 2
NS = 16
NW = NC * NS
CHUNK = 128
NCHUNK = 80
E_PAD = NW * NCHUNK * CHUNK
NCHUNK0 = 2 * NCHUNK
RPT = 624
TAIL_BASE = NS * RPT
TAIL = N - TAIL_BASE
NACC = N + 16
ZROWS = 16


def _zero_acc(zbuf, acc, sid, base_row):
  zero = jnp.zeros((16,), jnp.float32)
  D = zbuf.shape[1]

  @pl.loop(0, ZROWS)
  def _(r):
    for j in range(D // 16):
      zbuf[r, pl.ds(j * 16, 16)] = zero

  @pl.loop(0, RPT // ZROWS)
  def _(b):
    pltpu.sync_copy(zbuf, acc.at[pl.ds(base_row + b * ZROWS, ZROWS)])

  @pl.when(sid == NS - 1)
  def _():
    pltpu.sync_copy(zbuf, acc.at[pl.ds(TAIL_BASE, ZROWS)])
    pltpu.sync_copy(zbuf, acc.at[pl.ds(TAIL_BASE + ZROWS, ZROWS)])


NB = 16


def _edge_pipeline(y_hbm, src_hbm, dst_hbm, tix, isrc, idst,
                   rows0, rows1, acc, sem0, sem1, nblk):
  @pl.loop(0, nblk)
  def _(blk):
    pltpu.sync_copy(src_hbm.at[tix, pl.ds(blk * NB, NB)], isrc)
    pltpu.sync_copy(dst_hbm.at[tix, pl.ds(blk * NB, NB)], idst)

    @pl.loop(0, NB)
    def _(c):
      pltpu.async_copy(y_hbm.at[isrc.at[c]], rows0, sem0).wait()
      pltpu.sync_copy(rows0, acc.at[idst.at[c]], add=True)


def _drain_acc(acc, out_hbm, base_row, sid):
  row_slc = pl.ds(base_row, RPT)
  tail_slc = pl.ds(TAIL_BASE, TAIL)
  pltpu.sync_copy(acc.at[row_slc], out_hbm.at[row_slc])

  @pl.when(sid == NS - 1)
  def _():
    pltpu.sync_copy(acc.at[tail_slc], out_hbm.at[tail_slc])


def _make_sc_segsum():
  mesh = plsc.VectorSubcoreMesh(core_axis_name="c", subcore_axis_name="s")
  out_type = (jax.ShapeDtypeStruct((N, D_HID), jnp.float32),
              jax.ShapeDtypeStruct((N, D_HID), jnp.float32))
  scratch = [
      pltpu.VMEM((NB, CHUNK), jnp.int32),
      pltpu.VMEM((NB, CHUNK), jnp.int32),
      pltpu.VMEM((CHUNK, D_HID), jnp.float32),
      pltpu.VMEM((CHUNK, D_HID), jnp.float32),
      pltpu.VMEM((ZROWS, D_HID), jnp.float32),
      pltpu.VMEM_SHARED((NACC, D_HID), jnp.float32),
      pltpu.SemaphoreType.DMA,
      pltpu.SemaphoreType.DMA,
  ]

  def body(y_hbm, src_hbm, dst_hbm, p0_hbm, p1_hbm,
           isrc, idst, rows0, rows1, zbuf, acc, sem0, sem1):
    cid = lax.axis_index("c")
    sid = lax.axis_index("s")
    wid = sid * NC + cid
    base_row = sid * RPT

    _zero_acc(zbuf, acc, sid, base_row)
    plsc.subcore_barrier()

    _edge_pipeline(y_hbm, src_hbm, dst_hbm, wid, isrc, idst,
                   rows0, rows1, acc, sem0, sem1, NCHUNK // NB)

    plsc.subcore_barrier()

    @pl.when(cid == 0)
    def _():
      _drain_acc(acc, p0_hbm, base_row, sid)

    @pl.when(cid == 1)
    def _():
      _drain_acc(acc, p1_hbm, base_row, sid)

  return pl.kernel(body, out_type=out_type, mesh=mesh, scratch_types=scratch)


def _make_deg_hist():
  mesh = plsc.VectorSubcoreMesh(core_axis_name="c", subcore_axis_name="s")
  out_type = jax.ShapeDtypeStruct((NW, 1, NACC), jnp.float32)
  scratch = [
      pltpu.VMEM((NB, CHUNK), jnp.int32),
      pltpu.VMEM((1, NACC), jnp.float32),
  ]

  def body(dst_hbm, dh_hbm, idst, hist):
    cid = lax.axis_index("c")
    sid = lax.axis_index("s")
    wid = sid * NC + cid
    zero = jnp.zeros((16,), jnp.float32)
    ones_f = jnp.ones((16,), jnp.float32)
    zeros_i = jnp.zeros((16,), jnp.int32)

    @pl.loop(0, NACC // 16)
    def _(k):
      hist[0, pl.ds(k * 16, 16)] = zero

    @pl.loop(0, NCHUNK // NB)
    def _(blk):
      pltpu.sync_copy(dst_hbm.at[wid, pl.ds(blk * NB, NB)], idst)

      @pl.loop(0, NB)
      def _(r):
        for j in range(CHUNK // 16):
          v = idst[r, pl.ds(j * 16, 16)]
          plsc.addupdate_scatter(hist, [zeros_i, v], ones_f)

    pltpu.sync_copy(hist, dh_hbm.at[wid])

  cp = pltpu.CompilerParams()
  if "needs_layout_passes" in pltpu.CompilerParams.__dataclass_fields__:
    cp = dataclasses.replace(cp, needs_layout_passes=False)
  return pl.kernel(body, out_type=out_type, mesh=mesh,
                   scratch_types=scratch, compiler_params=cp)


_sc_pass0 = _make_sc_segsum()
_sc_pass1 = _make_sc_segsum()
_sc_pass2 = _make_sc_segsum()
_sc_deg = _make_deg_hist()


BN = 1000
GRID = N // BN
_F32 = jnp.float32


def _row_spec(d):
  return pl.BlockSpec((BN, d), lambda i: (i, 0))


def _full_spec(r, c):
  return pl.BlockSpec((r, c), lambda i: (0, 0))


def _dot(a, b):
  return jnp.dot(a, b, preferred_element_type=_F32)


def _stage_a(x, emb, wr, wn, wp, b, y_o, root_o):
  xv = x[...]
  y_o[...] = _dot(xv, wn[...])
  root_o[...] = _dot(xv, wr[...]) + _dot(emb[...], wp[...]) + b[...]


def _stage_b(p0, p1, dh, root, emb, wr, wn, wp, b, y_o, root_o, recip_o):
  deg = jnp.sum(dh[...], axis=1)
  rc = (1.0 / jnp.maximum(deg, 1.0))[:, None]
  recip_o[...] = jnp.broadcast_to(rc, (BN, 16))
  h = jnp.maximum(root[...] + (p0[...] + p1[...]) * rc, 0.0)
  y_o[...] = _dot(h, wn[...])
  root_o[...] = _dot(h, wr[...]) + _dot(emb[...], wp[...]) + b[...]


def _stage_c(p0, p1, recip, root, emb, wr, wn, wp, b, y_o, root_o):
  h = jnp.maximum(root[...] + (p0[...] + p1[...]) * recip[:, :1], 0.0)
  y_o[:, :D_OUT] = _dot(h, wn[...])
  y_o[:, D_OUT:] = jnp.zeros((BN, D_HID - D_OUT), _F32)
  root_o[...] = _dot(h, wr[...]) + _dot(emb[...], wp[...]) + b[...]


def _stage_d(p0, p1, recip, root, out_o):
  out_o[...] = root[...] + (p0[:, :D_OUT] + p1[:, :D_OUT]) * recip[:, :1]


def _tc_stage_a(x, emb, wr, wn, wp, b):
  return pl.pallas_call(
      _stage_a,
      grid=(GRID,),
      in_specs=[_row_spec(D_IN), _row_spec(D_PE),
                _full_spec(D_IN, D_HID), _full_spec(D_IN, D_HID),
                _full_spec(D_PE, D_HID), _full_spec(1, D_HID)],
      out_specs=[_row_spec(D_HID), _row_spec(D_HID)],
      out_shape=[jax.ShapeDtypeStruct((N, D_HID), _F32)] * 2,
  )(x, emb, wr, wn, wp, b)


def _tc_stage_b(p0, p1, dh, root, emb, wr, wn, wp, b):
  return pl.pallas_call(
      _stage_b,
      grid=(GRID,),
      in_specs=[_row_spec(D_HID), _row_spec(D_HID),
                pl.BlockSpec((BN, NW), lambda i: (i, 0)),
                _row_spec(D_HID), _row_spec(D_PE),
                _full_spec(D_HID, D_HID), _full_spec(D_HID, D_HID),
                _full_spec(D_PE, D_HID), _full_spec(1, D_HID)],
      out_specs=[_row_spec(D_HID), _row_spec(D_HID), _row_spec(16)],
      out_shape=[jax.ShapeDtypeStruct((N, D_HID), _F32),
                 jax.ShapeDtypeStruct((N, D_HID), _F32),
                 jax.ShapeDtypeStruct((N, 16), _F32)],
  )(p0, p1, dh, root, emb, wr, wn, wp, b)


def _tc_stage_c(p0, p1, recip, root, emb, wr, wn, wp, b):
  return pl.pallas_call(
      _stage_c,
      grid=(GRID,),
      in_specs=[_row_spec(D_HID), _row_spec(D_HID), _row_spec(16),
                _row_spec(D_HID), _row_spec(D_PE),
                _full_spec(D_HID, D_OUT), _full_spec(D_HID, D_OUT),
                _full_spec(D_PE, D_OUT), _full_spec(1, D_OUT)],
      out_specs=[_row_spec(D_HID), _row_spec(D_OUT)],
      out_shape=[jax.ShapeDtypeStruct((N, D_HID), _F32),
                 jax.ShapeDtypeStruct((N, D_OUT), _F32)],
  )(p0, p1, recip, root, emb, wr, wn, wp, b)


def _tc_stage_d(p0, p1, recip, root):
  return pl.pallas_call(
      _stage_d,
      grid=(GRID,),
      in_specs=[_row_spec(D_HID), _row_spec(D_HID), _row_spec(16),
                _row_spec(D_OUT)],
      out_specs=_row_spec(D_OUT),
      out_shape=jax.ShapeDtypeStruct((N, D_OUT), _F32),
  )(p0, p1, recip, root)


def kernel(x, adj_t, embeddings, Wr0, Wn0, Wp0, b0,
           Wr1, Wn1, Wp1, b1, Wr2, Wn2, Wp2, b2):
  b0r = b0.reshape(1, D_HID)
  b1r = b1.reshape(1, D_HID)
  b2r = b2.reshape(1, D_OUT)

  pad = E_PAD - E
  srcp = jnp.pad(adj_t[0], (0, pad))
  dstp = jnp.pad(adj_t[1], (0, pad), constant_values=N)
  src3 = srcp.reshape(NW, NCHUNK, CHUNK)
  dst3 = dstp.reshape(NW, NCHUNK, CHUNK)

  dh3 = _sc_deg(dst3)
  dh = dh3.reshape(NW, NACC)[:, :N].T
  y0, root0 = _tc_stage_a(x, embeddings, Wr0, Wn0, Wp0, b0r)
  p0, p1 = _sc_pass0(y0, src3, dst3)
  y1, root1, recip = _tc_stage_b(p0, p1, dh, root0, embeddings,
                                 Wr1, Wn1, Wp1, b1r)
  q0, q1 = _sc_pass1(y1, src3, dst3)
  y2, root2 = _tc_stage_c(q0, q1, recip, root1, embeddings,
                          Wr2, Wn2, Wp2, b2r)
  s0, s1 = _sc_pass2(y2, src3, dst3)
  return _tc_stage_d(s0, s1, recip, root2)


# speedup vs baseline: 1.7229x; 1.7229x over previous
import dataclasses
import functools

import jax
import jax.numpy as jnp
from jax import lax
from jax.experimental import pallas as pl
from jax.experimental.pallas import tpu as pltpu
from jax.experimental.pallas import tpu_sc as plsc

N = 10000
E = 320000
D_IN = 128
D_HID = 128
D_OUT = 64
D_PE = 128

NC = 2
NS = 16
NW = NC * NS
CHUNK = 80
NCHUNK = E // (NW * CHUNK)
RPT = 624
TAIL_BASE = NS * RPT
TAIL = N - TAIL_BASE
NACC = N + 16
ZROWS = 16


def _zero_acc(zbuf, acc, sid, base_row):
  zero = jnp.zeros((16,), jnp.float32)
  D = zbuf.shape[1]

  @pl.loop(0, ZROWS)
  def _(r):
    for j in range(D // 16):
      zbuf[r, pl.ds(j * 16, 16)] = zero

  @pl.loop(0, RPT // ZROWS)
  def _(b):
    pltpu.sync_copy(zbuf, acc.at[pl.ds(base_row + b * ZROWS, ZROWS)])

  @pl.when(sid == NS - 1)
  def _():
    pltpu.sync_copy(zbuf, acc.at[pl.ds(TAIL_BASE, ZROWS)])
    pltpu.sync_copy(zbuf, acc.at[pl.ds(TAIL_BASE + ZROWS, ZROWS)])


def _edge_pipeline(y_hbm, src_hbm, dst_hbm, tix, srcv, dstv,
                   rows, acc, sem):
  ebase = tix * NCHUNK * CHUNK

  @pl.loop(0, NCHUNK)
  def _(c):
    off = ebase + c * CHUNK
    pltpu.sync_copy(src_hbm.at[pl.ds(off, CHUNK)], srcv)
    pltpu.sync_copy(dst_hbm.at[pl.ds(off, CHUNK)], dstv)
    pltpu.async_copy(y_hbm.at[srcv], rows, sem).wait()
    pltpu.sync_copy(rows, acc.at[dstv], add=True)


def _drain_acc(acc, out_hbm, base_row, sid):
  row_slc = pl.ds(base_row, RPT)
  tail_slc = pl.ds(TAIL_BASE, TAIL)
  pltpu.sync_copy(acc.at[row_slc], out_hbm.at[row_slc])

  @pl.when(sid == NS - 1)
  def _():
    pltpu.sync_copy(acc.at[tail_slc], out_hbm.at[tail_slc])


def _make_sc_segsum():
  mesh = plsc.VectorSubcoreMesh(core_axis_name="c", subcore_axis_name="s")
  out_type = (jax.ShapeDtypeStruct((N, D_HID), jnp.float32),
              jax.ShapeDtypeStruct((N, D_HID), jnp.float32))
  scratch = [
      pltpu.VMEM((CHUNK,), jnp.int32),
      pltpu.VMEM((CHUNK,), jnp.int32),
      pltpu.VMEM((CHUNK, D_HID), jnp.float32),
      pltpu.VMEM((ZROWS, D_HID), jnp.float32),
      pltpu.VMEM_SHARED((NACC, D_HID), jnp.float32),
      pltpu.SemaphoreType.DMA,
  ]

  def body(y_hbm, src_hbm, dst_hbm, p0_hbm, p1_hbm,
           srcv, dstv, rows, zbuf, acc, sem):
    cid = lax.axis_index("c")
    sid = lax.axis_index("s")
    wid = sid * NC + cid
    base_row = sid * RPT

    _zero_acc(zbuf, acc, sid, base_row)
    plsc.subcore_barrier()

    _edge_pipeline(y_hbm, src_hbm, dst_hbm, wid, srcv, dstv,
                   rows, acc, sem)

    plsc.subcore_barrier()

    @pl.when(cid == 0)
    def _():
      _drain_acc(acc, p0_hbm, base_row, sid)

    @pl.when(cid == 1)
    def _():
      _drain_acc(acc, p1_hbm, base_row, sid)

  return pl.kernel(body, out_type=out_type, mesh=mesh, scratch_types=scratch)


def _make_deg_hist():
  mesh = plsc.VectorSubcoreMesh(core_axis_name="c", subcore_axis_name="s")
  out_type = jax.ShapeDtypeStruct((NW, 1, NACC), jnp.float32)
  scratch = [
      pltpu.VMEM((NCHUNK, CHUNK), jnp.int32),
      pltpu.VMEM((1, NACC), jnp.float32),
  ]

  def body(dst_hbm, dh_hbm, idst, hist):
    cid = lax.axis_index("c")
    sid = lax.axis_index("s")
    wid = sid * NC + cid
    zero = jnp.zeros((16,), jnp.float32)
    ones_f = jnp.ones((16,), jnp.float32)
    zeros_i = jnp.zeros((16,), jnp.int32)

    @pl.loop(0, NACC // 16)
    def _(k):
      hist[0, pl.ds(k * 16, 16)] = zero

    pltpu.sync_copy(dst_hbm.at[wid], idst)

    @pl.loop(0, NCHUNK)
    def _(r):
      for j in range(CHUNK // 16):
        v = idst[r, pl.ds(j * 16, 16)]
        plsc.addupdate_scatter(hist, [zeros_i, v], ones_f)

    pltpu.sync_copy(hist, dh_hbm.at[wid])

  cp = pltpu.CompilerParams()
  if "needs_layout_passes" in pltpu.CompilerParams.__dataclass_fields__:
    cp = dataclasses.replace(cp, needs_layout_passes=False)
  return pl.kernel(body, out_type=out_type, mesh=mesh,
                   scratch_types=scratch, compiler_params=cp)


_sc_pass0 = _make_sc_segsum()
_sc_pass1 = _make_sc_segsum()
_sc_pass2 = _make_sc_segsum()
_sc_deg = _make_deg_hist()


BN = 1000
GRID = N // BN
_F32 = jnp.float32


def _row_spec(d):
  return pl.BlockSpec((BN, d), lambda i: (i, 0))


def _full_spec(r, c):
  return pl.BlockSpec((r, c), lambda i: (0, 0))


def _dot(a, b):
  return jnp.dot(a, b, preferred_element_type=_F32)


def _stage_a(x, emb, wr, wn, wp, b, y_o, root_o):
  xv = x[...]
  y_o[...] = _dot(xv, wn[...])
  root_o[...] = _dot(xv, wr[...]) + _dot(emb[...], wp[...]) + b[...]


def _stage_b(p0, p1, dh, root, emb, wr, wn, wp, b, y_o, root_o, recip_o):
  deg = jnp.sum(dh[...], axis=1)
  rc = (1.0 / jnp.maximum(deg, 1.0))[:, None]
  recip_o[...] = jnp.broadcast_to(rc, (BN, 16))
  h = jnp.maximum(root[...] + (p0[...] + p1[...]) * rc, 0.0)
  y_o[...] = _dot(h, wn[...])
  root_o[...] = _dot(h, wr[...]) + _dot(emb[...], wp[...]) + b[...]


def _stage_c(p0, p1, recip, root, emb, wr, wn, wp, b, y_o, root_o):
  h = jnp.maximum(root[...] + (p0[...] + p1[...]) * recip[:, :1], 0.0)
  y_o[:, :D_OUT] = _dot(h, wn[...])
  y_o[:, D_OUT:] = jnp.zeros((BN, D_HID - D_OUT), _F32)
  root_o[...] = _dot(h, wr[...]) + _dot(emb[...], wp[...]) + b[...]


def _stage_d(p0, p1, recip, root, out_o):
  out_o[...] = root[...] + (p0[:, :D_OUT] + p1[:, :D_OUT]) * recip[:, :1]


def _tc_stage_a(x, emb, wr, wn, wp, b):
  return pl.pallas_call(
      _stage_a,
      grid=(GRID,),
      in_specs=[_row_spec(D_IN), _row_spec(D_PE),
                _full_spec(D_IN, D_HID), _full_spec(D_IN, D_HID),
                _full_spec(D_PE, D_HID), _full_spec(1, D_HID)],
      out_specs=[_row_spec(D_HID), _row_spec(D_HID)],
      out_shape=[jax.ShapeDtypeStruct((N, D_HID), _F32)] * 2,
  )(x, emb, wr, wn, wp, b)


def _tc_stage_b(p0, p1, dh, root, emb, wr, wn, wp, b):
  return pl.pallas_call(
      _stage_b,
      grid=(GRID,),
      in_specs=[_row_spec(D_HID), _row_spec(D_HID),
                pl.BlockSpec((BN, NW), lambda i: (i, 0)),
                _row_spec(D_HID), _row_spec(D_PE),
                _full_spec(D_HID, D_HID), _full_spec(D_HID, D_HID),
                _full_spec(D_PE, D_HID), _full_spec(1, D_HID)],
      out_specs=[_row_spec(D_HID), _row_spec(D_HID), _row_spec(16)],
      out_shape=[jax.ShapeDtypeStruct((N, D_HID), _F32),
                 jax.ShapeDtypeStruct((N, D_HID), _F32),
                 jax.ShapeDtypeStruct((N, 16), _F32)],
  )(p0, p1, dh, root, emb, wr, wn, wp, b)


def _tc_stage_c(p0, p1, recip, root, emb, wr, wn, wp, b):
  return pl.pallas_call(
      _stage_c,
      grid=(GRID,),
      in_specs=[_row_spec(D_HID), _row_spec(D_HID), _row_spec(16),
                _row_spec(D_HID), _row_spec(D_PE),
                _full_spec(D_HID, D_OUT), _full_spec(D_HID, D_OUT),
                _full_spec(D_PE, D_OUT), _full_spec(1, D_OUT)],
      out_specs=[_row_spec(D_HID), _row_spec(D_OUT)],
      out_shape=[jax.ShapeDtypeStruct((N, D_HID), _F32),
                 jax.ShapeDtypeStruct((N, D_OUT), _F32)],
  )(p0, p1, recip, root, emb, wr, wn, wp, b)


def _tc_stage_d(p0, p1, recip, root):
  return pl.pallas_call(
      _stage_d,
      grid=(GRID,),
      in_specs=[_row_spec(D_HID), _row_spec(D_HID), _row_spec(16),
                _row_spec(D_OUT)],
      out_specs=_row_spec(D_OUT),
      out_shape=jax.ShapeDtypeStruct((N, D_OUT), _F32),
  )(p0, p1, recip, root)


def kernel(x, adj_t, embeddings, Wr0, Wn0, Wp0, b0,
           Wr1, Wn1, Wp1, b1, Wr2, Wn2, Wp2, b2):
  b0r = b0.reshape(1, D_HID)
  b1r = b1.reshape(1, D_HID)
  b2r = b2.reshape(1, D_OUT)

  src = adj_t[0]
  dst = adj_t[1]
  dst3 = dst.reshape(NW, NCHUNK, CHUNK)

  dh3 = _sc_deg(dst3)
  dh = dh3.reshape(NW, NACC)[:, :N].T
  y0, root0 = _tc_stage_a(x, embeddings, Wr0, Wn0, Wp0, b0r)
  p0, p1 = _sc_pass0(y0, src, dst)
  y1, root1, recip = _tc_stage_b(p0, p1, dh, root0, embeddings,
                                 Wr1, Wn1, Wp1, b1r)
  q0, q1 = _sc_pass1(y1, src, dst)
  y2, root2 = _tc_stage_c(q0, q1, recip, root1, embeddings,
                          Wr2, Wn2, Wp2, b2r)
  s0, s1 = _sc_pass2(y2, src, dst)
  return _tc_stage_d(s0, s1, recip, root2)


# speedup vs baseline: 2.7407x; 1.5907x over previous
import dataclasses
import functools

import jax
import jax.numpy as jnp
from jax import lax
from jax.experimental import pallas as pl
from jax.experimental.pallas import tpu as pltpu
from jax.experimental.pallas import tpu_sc as plsc

N = 10000
E = 320000
D_IN = 128
D_HID = 128
D_OUT = 64
D_PE = 128

NC = 2
NS = 16
NW = NC * NS
CHUNK = 80
NCHUNK = E // (NW * CHUNK)
RPT = 624
TAIL_BASE = NS * RPT
TAIL = N - TAIL_BASE
NACC = N + 16
ZROWS = 16


def _zero_acc(zbuf, acc, sid, base_row):
  zero = jnp.zeros((16,), jnp.float32)
  D = zbuf.shape[1]

  @pl.loop(0, ZROWS)
  def _(r):
    for j in range(D // 16):
      zbuf[r, pl.ds(j * 16, 16)] = zero

  @pl.loop(0, RPT // ZROWS)
  def _(b):
    pltpu.sync_copy(zbuf, acc.at[pl.ds(base_row + b * ZROWS, ZROWS)])

  @pl.when(sid == NS - 1)
  def _():
    pltpu.sync_copy(zbuf, acc.at[pl.ds(TAIL_BASE, ZROWS)])
    pltpu.sync_copy(zbuf, acc.at[pl.ds(TAIL_BASE + ZROWS, ZROWS)])


def _edge_pipeline(y_hbm, src_hbm, dst_hbm, tix, srcv0, dstv0, srcv1, dstv1,
                   rows0, rows1, acc, sem0, sem1):
  ebase = tix * NCHUNK * CHUNK

  def load_idx(c, sv, dv):
    off = ebase + c * CHUNK
    pltpu.sync_copy(src_hbm.at[pl.ds(off, CHUNK)], sv)
    pltpu.sync_copy(dst_hbm.at[pl.ds(off, CHUNK)], dv)

  load_idx(0, srcv0, dstv0)
  pltpu.async_copy(y_hbm.at[srcv0], rows0, sem0)

  @pl.loop(0, NCHUNK // 2)
  def _(p):
    c0 = 2 * p
    load_idx(c0 + 1, srcv1, dstv1)
    pltpu.async_copy(y_hbm.at[srcv1], rows1, sem1)
    pltpu.make_async_copy(y_hbm.at[srcv0], rows0, sem0).wait()
    pltpu.sync_copy(rows0, acc.at[dstv0], add=True)

    @pl.when(c0 + 2 < NCHUNK)
    def _():
      load_idx(c0 + 2, srcv0, dstv0)
      pltpu.async_copy(y_hbm.at[srcv0], rows0, sem0)

    pltpu.make_async_copy(y_hbm.at[srcv1], rows1, sem1).wait()
    pltpu.sync_copy(rows1, acc.at[dstv1], add=True)

  if NCHUNK % 2:
    pltpu.make_async_copy(y_hbm.at[srcv0], rows0, sem0).wait()
    pltpu.sync_copy(rows0, acc.at[dstv0], add=True)


def _drain_acc(acc, out_hbm, base_row, sid):
  row_slc = pl.ds(base_row, RPT)
  tail_slc = pl.ds(TAIL_BASE, TAIL)
  pltpu.sync_copy(acc.at[row_slc], out_hbm.at[row_slc])

  @pl.when(sid == NS - 1)
  def _():
    pltpu.sync_copy(acc.at[tail_slc], out_hbm.at[tail_slc])


def _make_sc_segsum():
  mesh = plsc.VectorSubcoreMesh(core_axis_name="c", subcore_axis_name="s")
  out_type = (jax.ShapeDtypeStruct((N, D_HID), jnp.float32),
              jax.ShapeDtypeStruct((N, D_HID), jnp.float32))
  scratch = [
      pltpu.VMEM((CHUNK,), jnp.int32),
      pltpu.VMEM((CHUNK,), jnp.int32),
      pltpu.VMEM((CHUNK,), jnp.int32),
      pltpu.VMEM((CHUNK,), jnp.int32),
      pltpu.VMEM((CHUNK, D_HID), jnp.float32),
      pltpu.VMEM((CHUNK, D_HID), jnp.float32),
      pltpu.VMEM((ZROWS, D_HID), jnp.float32),
      pltpu.VMEM_SHARED((NACC, D_HID), jnp.float32),
      pltpu.SemaphoreType.DMA,
      pltpu.SemaphoreType.DMA,
  ]

  def body(y_hbm, src_hbm, dst_hbm, p0_hbm, p1_hbm,
           srcv0, dstv0, srcv1, dstv1, rows0, rows1, zbuf, acc, sem0, sem1):
    cid = lax.axis_index("c")
    sid = lax.axis_index("s")
    wid = sid * NC + cid
    base_row = sid * RPT

    _zero_acc(zbuf, acc, sid, base_row)
    plsc.subcore_barrier()

    _edge_pipeline(y_hbm, src_hbm, dst_hbm, wid, srcv0, dstv0, srcv1, dstv1,
                   rows0, rows1, acc, sem0, sem1)

    plsc.subcore_barrier()

    @pl.when(cid == 0)
    def _():
      _drain_acc(acc, p0_hbm, base_row, sid)

    @pl.when(cid == 1)
    def _():
      _drain_acc(acc, p1_hbm, base_row, sid)

  return pl.kernel(body, out_type=out_type, mesh=mesh, scratch_types=scratch)


def _make_deg_hist():
  mesh = plsc.VectorSubcoreMesh(core_axis_name="c", subcore_axis_name="s")
  out_type = jax.ShapeDtypeStruct((NW, 1, NACC), jnp.float32)
  scratch = [
      pltpu.VMEM((NCHUNK, CHUNK), jnp.int32),
      pltpu.VMEM((1, NACC), jnp.float32),
  ]

  def body(dst_hbm, dh_hbm, idst, hist):
    cid = lax.axis_index("c")
    sid = lax.axis_index("s")
    wid = sid * NC + cid
    zero = jnp.zeros((16,), jnp.float32)
    ones_f = jnp.ones((16,), jnp.float32)
    zeros_i = jnp.zeros((16,), jnp.int32)

    @pl.loop(0, NACC // 16)
    def _(k):
      hist[0, pl.ds(k * 16, 16)] = zero

    pltpu.sync_copy(dst_hbm.at[wid], idst)

    @pl.loop(0, NCHUNK)
    def _(r):
      for j in range(CHUNK // 16):
        v = idst[r, pl.ds(j * 16, 16)]
        plsc.addupdate_scatter(hist, [zeros_i, v], ones_f)

    pltpu.sync_copy(hist, dh_hbm.at[wid])

  cp = pltpu.CompilerParams()
  if "needs_layout_passes" in pltpu.CompilerParams.__dataclass_fields__:
    cp = dataclasses.replace(cp, needs_layout_passes=False)
  return pl.kernel(body, out_type=out_type, mesh=mesh,
                   scratch_types=scratch, compiler_params=cp)


_sc_pass0 = _make_sc_segsum()
_sc_pass1 = _make_sc_segsum()
_sc_pass2 = _make_sc_segsum()
_sc_deg = _make_deg_hist()


BN = 1000
GRID = N // BN
_F32 = jnp.float32


def _row_spec(d):
  return pl.BlockSpec((BN, d), lambda i: (i, 0))


def _full_spec(r, c):
  return pl.BlockSpec((r, c), lambda i: (0, 0))


def _dot(a, b):
  return jnp.dot(a, b, preferred_element_type=_F32)


def _stage_a(x, emb, wr, wn, wp, b, y_o, root_o):
  xv = x[...]
  y_o[...] = _dot(xv, wn[...])
  root_o[...] = _dot(xv, wr[...]) + _dot(emb[...], wp[...]) + b[...]


def _stage_b(p0, p1, dh, root, emb, wr, wn, wp, b, y_o, root_o, recip_o):
  deg = jnp.sum(dh[...], axis=1)
  rc = (1.0 / jnp.maximum(deg, 1.0))[:, None]
  recip_o[...] = jnp.broadcast_to(rc, (BN, 16))
  h = jnp.maximum(root[...] + (p0[...] + p1[...]) * rc, 0.0)
  y_o[...] = _dot(h, wn[...])
  root_o[...] = _dot(h, wr[...]) + _dot(emb[...], wp[...]) + b[...]


def _stage_c(p0, p1, recip, root, emb, wr, wn, wp, b, y_o, root_o):
  h = jnp.maximum(root[...] + (p0[...] + p1[...]) * recip[:, :1], 0.0)
  y_o[:, :D_OUT] = _dot(h, wn[...])
  y_o[:, D_OUT:] = jnp.zeros((BN, D_HID - D_OUT), _F32)
  root_o[...] = _dot(h, wr[...]) + _dot(emb[...], wp[...]) + b[...]


def _stage_d(p0, p1, recip, root, out_o):
  out_o[...] = root[...] + (p0[:, :D_OUT] + p1[:, :D_OUT]) * recip[:, :1]


def _tc_stage_a(x, emb, wr, wn, wp, b):
  return pl.pallas_call(
      _stage_a,
      grid=(GRID,),
      in_specs=[_row_spec(D_IN), _row_spec(D_PE),
                _full_spec(D_IN, D_HID), _full_spec(D_IN, D_HID),
                _full_spec(D_PE, D_HID), _full_spec(1, D_HID)],
      out_specs=[_row_spec(D_HID), _row_spec(D_HID)],
      out_shape=[jax.ShapeDtypeStruct((N, D_HID), _F32)] * 2,
  )(x, emb, wr, wn, wp, b)


def _tc_stage_b(p0, p1, dh, root, emb, wr, wn, wp, b):
  return pl.pallas_call(
      _stage_b,
      grid=(GRID,),
      in_specs=[_row_spec(D_HID), _row_spec(D_HID),
                pl.BlockSpec((BN, NW), lambda i: (i, 0)),
                _row_spec(D_HID), _row_spec(D_PE),
                _full_spec(D_HID, D_HID), _full_spec(D_HID, D_HID),
                _full_spec(D_PE, D_HID), _full_spec(1, D_HID)],
      out_specs=[_row_spec(D_HID), _row_spec(D_HID), _row_spec(16)],
      out_shape=[jax.ShapeDtypeStruct((N, D_HID), _F32),
                 jax.ShapeDtypeStruct((N, D_HID), _F32),
                 jax.ShapeDtypeStruct((N, 16), _F32)],
  )(p0, p1, dh, root, emb, wr, wn, wp, b)


def _tc_stage_c(p0, p1, recip, root, emb, wr, wn, wp, b):
  return pl.pallas_call(
      _stage_c,
      grid=(GRID,),
      in_specs=[_row_spec(D_HID), _row_spec(D_HID), _row_spec(16),
                _row_spec(D_HID), _row_spec(D_PE),
                _full_spec(D_HID, D_OUT), _full_spec(D_HID, D_OUT),
                _full_spec(D_PE, D_OUT), _full_spec(1, D_OUT)],
      out_specs=[_row_spec(D_HID), _row_spec(D_OUT)],
      out_shape=[jax.ShapeDtypeStruct((N, D_HID), _F32),
                 jax.ShapeDtypeStruct((N, D_OUT), _F32)],
  )(p0, p1, recip, root, emb, wr, wn, wp, b)


def _tc_stage_d(p0, p1, recip, root):
  return pl.pallas_call(
      _stage_d,
      grid=(GRID,),
      in_specs=[_row_spec(D_HID), _row_spec(D_HID), _row_spec(16),
                _row_spec(D_OUT)],
      out_specs=_row_spec(D_OUT),
      out_shape=jax.ShapeDtypeStruct((N, D_OUT), _F32),
  )(p0, p1, recip, root)


def kernel(x, adj_t, embeddings, Wr0, Wn0, Wp0, b0,
           Wr1, Wn1, Wp1, b1, Wr2, Wn2, Wp2, b2):
  b0r = b0.reshape(1, D_HID)
  b1r = b1.reshape(1, D_HID)
  b2r = b2.reshape(1, D_OUT)

  src = adj_t[0]
  dst = adj_t[1]
  dst3 = dst.reshape(NW, NCHUNK, CHUNK)

  dh3 = _sc_deg(dst3)
  dh = dh3.reshape(NW, NACC)[:, :N].T
  y0, root0 = _tc_stage_a(x, embeddings, Wr0, Wn0, Wp0, b0r)
  p0, p1 = _sc_pass0(y0, src, dst)
  y1, root1, recip = _tc_stage_b(p0, p1, dh, root0, embeddings,
                                 Wr1, Wn1, Wp1, b1r)
  q0, q1 = _sc_pass1(y1, src, dst)
  y2, root2 = _tc_stage_c(q0, q1, recip, root1, embeddings,
                          Wr2, Wn2, Wp2, b2r)
  s0, s1 = _sc_pass2(y2, src, dst)
  return _tc_stage_d(s0, s1, recip, root2)


# speedup vs baseline: 2.7512x; 1.0038x over previous
import dataclasses
import functools

import jax
import jax.numpy as jnp
from jax import lax
from jax.experimental import pallas as pl
from jax.experimental.pallas import tpu as pltpu
from jax.experimental.pallas import tpu_sc as plsc

N = 10000
E = 320000
D_IN = 128
D_HID = 128
D_OUT = 64
D_PE = 128

NC = 2
NS = 16
NW = NC * NS
CHUNK = 80
NCHUNK = E // (NW * CHUNK)
RPT = 624
TAIL_BASE = NS * RPT
TAIL = N - TAIL_BASE
NACC = N + 16
ZROWS = 16


def _zero_acc(zbuf, acc, sid, base_row):
  zero = jnp.zeros((16,), jnp.float32)
  D = zbuf.shape[1]

  @pl.loop(0, ZROWS)
  def _(r):
    for j in range(D // 16):
      zbuf[r, pl.ds(j * 16, 16)] = zero

  @pl.loop(0, RPT // ZROWS)
  def _(b):
    pltpu.sync_copy(zbuf, acc.at[pl.ds(base_row + b * ZROWS, ZROWS)])

  @pl.when(sid == NS - 1)
  def _():
    pltpu.sync_copy(zbuf, acc.at[pl.ds(TAIL_BASE, ZROWS)])
    pltpu.sync_copy(zbuf, acc.at[pl.ds(TAIL_BASE + ZROWS, ZROWS)])


def _edge_pipeline(y_hbm, src_hbm, dst_hbm, tix, srcv0, dstv0, srcv1, dstv1,
                   srcv2, dstv2, rows0, rows1, rows2, acc, sem0, sem1, sem2):
  ebase = tix * NCHUNK * CHUNK
  bufs = ((srcv0, dstv0, rows0, sem0),
          (srcv1, dstv1, rows1, sem1),
          (srcv2, dstv2, rows2, sem2))

  def load_idx(c, sv, dv):
    off = ebase + c * CHUNK
    pltpu.sync_copy(src_hbm.at[pl.ds(off, CHUNK)], sv)
    pltpu.sync_copy(dst_hbm.at[pl.ds(off, CHUNK)], dv)

  def issue(c, b):
    sv, dv, rw, sm = bufs[b]
    load_idx(c, sv, dv)
    pltpu.async_copy(y_hbm.at[sv], rw, sm)

  def wait_scatter(b):
    sv, dv, rw, sm = bufs[b]
    pltpu.make_async_copy(y_hbm.at[sv], rw, sm).wait()
    pltpu.sync_copy(rw, acc.at[dv], add=True)

  K = NCHUNK // 3
  issue(0, 0)
  issue(1, 1)

  @pl.loop(0, K)
  def _(p):
    c0 = 3 * p
    issue(c0 + 2, 2)
    wait_scatter(0)

    @pl.when(c0 + 3 < NCHUNK)
    def _():
      issue(c0 + 3, 0)

    wait_scatter(1)

    @pl.when(c0 + 4 < NCHUNK)
    def _():
      issue(c0 + 4, 1)

    wait_scatter(2)

  for c in range(3 * K, NCHUNK):
    wait_scatter(c % 3)


def _drain_acc(acc, out_hbm, base_row, sid):
  row_slc = pl.ds(base_row, RPT)
  tail_slc = pl.ds(TAIL_BASE, TAIL)
  pltpu.sync_copy(acc.at[row_slc], out_hbm.at[row_slc])

  @pl.when(sid == NS - 1)
  def _():
    pltpu.sync_copy(acc.at[tail_slc], out_hbm.at[tail_slc])


def _make_sc_segsum():
  mesh = plsc.VectorSubcoreMesh(core_axis_name="c", subcore_axis_name="s")
  out_type = (jax.ShapeDtypeStruct((N, D_HID), jnp.float32),
              jax.ShapeDtypeStruct((N, D_HID), jnp.float32))
  scratch = [
      pltpu.VMEM((CHUNK,), jnp.int32),
      pltpu.VMEM((CHUNK,), jnp.int32),
      pltpu.VMEM((CHUNK,), jnp.int32),
      pltpu.VMEM((CHUNK,), jnp.int32),
      pltpu.VMEM((CHUNK,), jnp.int32),
      pltpu.VMEM((CHUNK,), jnp.int32),
      pltpu.VMEM((CHUNK, D_HID), jnp.float32),
      pltpu.VMEM((CHUNK, D_HID), jnp.float32),
      pltpu.VMEM((CHUNK, D_HID), jnp.float32),
      pltpu.VMEM((ZROWS, D_HID), jnp.float32),
      pltpu.VMEM_SHARED((NACC, D_HID), jnp.float32),
      pltpu.SemaphoreType.DMA,
      pltpu.SemaphoreType.DMA,
      pltpu.SemaphoreType.DMA,
  ]

  def body(y_hbm, src_hbm, dst_hbm, p0_hbm, p1_hbm,
           srcv0, dstv0, srcv1, dstv1, srcv2, dstv2,
           rows0, rows1, rows2, zbuf, acc, sem0, sem1, sem2):
    cid = lax.axis_index("c")
    sid = lax.axis_index("s")
    wid = sid * NC + cid
    base_row = sid * RPT

    _zero_acc(zbuf, acc, sid, base_row)
    plsc.subcore_barrier()

    _edge_pipeline(y_hbm, src_hbm, dst_hbm, wid, srcv0, dstv0, srcv1, dstv1,
                   srcv2, dstv2, rows0, rows1, rows2, acc, sem0, sem1, sem2)

    plsc.subcore_barrier()

    @pl.when(cid == 0)
    def _():
      _drain_acc(acc, p0_hbm, base_row, sid)

    @pl.when(cid == 1)
    def _():
      _drain_acc(acc, p1_hbm, base_row, sid)

  return pl.kernel(body, out_type=out_type, mesh=mesh, scratch_types=scratch)


def _make_deg_hist():
  mesh = plsc.VectorSubcoreMesh(core_axis_name="c", subcore_axis_name="s")
  out_type = jax.ShapeDtypeStruct((NW, 1, NACC), jnp.float32)
  scratch = [
      pltpu.VMEM((NCHUNK, CHUNK), jnp.int32),
      pltpu.VMEM((1, NACC), jnp.float32),
  ]

  def body(dst_hbm, dh_hbm, idst, hist):
    cid = lax.axis_index("c")
    sid = lax.axis_index("s")
    wid = sid * NC + cid
    zero = jnp.zeros((16,), jnp.float32)
    ones_f = jnp.ones((16,), jnp.float32)
    zeros_i = jnp.zeros((16,), jnp.int32)

    @pl.loop(0, NACC // 16)
    def _(k):
      hist[0, pl.ds(k * 16, 16)] = zero

    pltpu.sync_copy(dst_hbm.at[wid], idst)

    @pl.loop(0, NCHUNK)
    def _(r):
      for j in range(CHUNK // 16):
        v = idst[r, pl.ds(j * 16, 16)]
        plsc.addupdate_scatter(hist, [zeros_i, v], ones_f)

    pltpu.sync_copy(hist, dh_hbm.at[wid])

  cp = pltpu.CompilerParams()
  if "needs_layout_passes" in pltpu.CompilerParams.__dataclass_fields__:
    cp = dataclasses.replace(cp, needs_layout_passes=False)
  return pl.kernel(body, out_type=out_type, mesh=mesh,
                   scratch_types=scratch, compiler_params=cp)


_sc_pass0 = _make_sc_segsum()
_sc_pass1 = _make_sc_segsum()
_sc_pass2 = _make_sc_segsum()
_sc_deg = _make_deg_hist()


BN = 1000
GRID = N // BN
_F32 = jnp.float32


def _row_spec(d):
  return pl.BlockSpec((BN, d), lambda i: (i, 0))


def _full_spec(r, c):
  return pl.BlockSpec((r, c), lambda i: (0, 0))


def _dot(a, b):
  return jnp.dot(a, b, preferred_element_type=_F32)


def _stage_a(x, emb, wr, wn, wp, b, y_o, root_o):
  xv = x[...]
  y_o[...] = _dot(xv, wn[...])
  root_o[...] = _dot(xv, wr[...]) + _dot(emb[...], wp[...]) + b[...]


def _stage_b(p0, p1, dh, root, emb, wr, wn, wp, b, y_o, root_o, recip_o):
  deg = jnp.sum(dh[...], axis=1)
  rc = (1.0 / jnp.maximum(deg, 1.0))[:, None]
  recip_o[...] = jnp.broadcast_to(rc, (BN, 16))
  h = jnp.maximum(root[...] + (p0[...] + p1[...]) * rc, 0.0)
  y_o[...] = _dot(h, wn[...])
  root_o[...] = _dot(h, wr[...]) + _dot(emb[...], wp[...]) + b[...]


def _stage_c(p0, p1, recip, root, emb, wr, wn, wp, b, y_o, root_o):
  h = jnp.maximum(root[...] + (p0[...] + p1[...]) * recip[:, :1], 0.0)
  y_o[:, :D_OUT] = _dot(h, wn[...])
  y_o[:, D_OUT:] = jnp.zeros((BN, D_HID - D_OUT), _F32)
  root_o[...] = _dot(h, wr[...]) + _dot(emb[...], wp[...]) + b[...]


def _stage_d(p0, p1, recip, root, out_o):
  out_o[...] = root[...] + (p0[:, :D_OUT] + p1[:, :D_OUT]) * recip[:, :1]


def _tc_stage_a(x, emb, wr, wn, wp, b):
  return pl.pallas_call(
      _stage_a,
      grid=(GRID,),
      in_specs=[_row_spec(D_IN), _row_spec(D_PE),
                _full_spec(D_IN, D_HID), _full_spec(D_IN, D_HID),
                _full_spec(D_PE, D_HID), _full_spec(1, D_HID)],
      out_specs=[_row_spec(D_HID), _row_spec(D_HID)],
      out_shape=[jax.ShapeDtypeStruct((N, D_HID), _F32)] * 2,
  )(x, emb, wr, wn, wp, b)


def _tc_stage_b(p0, p1, dh, root, emb, wr, wn, wp, b):
  return pl.pallas_call(
      _stage_b,
      grid=(GRID,),
      in_specs=[_row_spec(D_HID), _row_spec(D_HID),
                pl.BlockSpec((BN, NW), lambda i: (i, 0)),
                _row_spec(D_HID), _row_spec(D_PE),
                _full_spec(D_HID, D_HID), _full_spec(D_HID, D_HID),
                _full_spec(D_PE, D_HID), _full_spec(1, D_HID)],
      out_specs=[_row_spec(D_HID), _row_spec(D_HID), _row_spec(16)],
      out_shape=[jax.ShapeDtypeStruct((N, D_HID), _F32),
                 jax.ShapeDtypeStruct((N, D_HID), _F32),
                 jax.ShapeDtypeStruct((N, 16), _F32)],
  )(p0, p1, dh, root, emb, wr, wn, wp, b)


def _tc_stage_c(p0, p1, recip, root, emb, wr, wn, wp, b):
  return pl.pallas_call(
      _stage_c,
      grid=(GRID,),
      in_specs=[_row_spec(D_HID), _row_spec(D_HID), _row_spec(16),
                _row_spec(D_HID), _row_spec(D_PE),
                _full_spec(D_HID, D_OUT), _full_spec(D_HID, D_OUT),
                _full_spec(D_PE, D_OUT), _full_spec(1, D_OUT)],
      out_specs=[_row_spec(D_HID), _row_spec(D_OUT)],
      out_shape=[jax.ShapeDtypeStruct((N, D_HID), _F32),
                 jax.ShapeDtypeStruct((N, D_OUT), _F32)],
  )(p0, p1, recip, root, emb, wr, wn, wp, b)


def _tc_stage_d(p0, p1, recip, root):
  return pl.pallas_call(
      _stage_d,
      grid=(GRID,),
      in_specs=[_row_spec(D_HID), _row_spec(D_HID), _row_spec(16),
                _row_spec(D_OUT)],
      out_specs=_row_spec(D_OUT),
      out_shape=jax.ShapeDtypeStruct((N, D_OUT), _F32),
  )(p0, p1, recip, root)


def kernel(x, adj_t, embeddings, Wr0, Wn0, Wp0, b0,
           Wr1, Wn1, Wp1, b1, Wr2, Wn2, Wp2, b2):
  b0r = b0.reshape(1, D_HID)
  b1r = b1.reshape(1, D_HID)
  b2r = b2.reshape(1, D_OUT)

  src = adj_t[0]
  dst = adj_t[1]
  dst3 = dst.reshape(NW, NCHUNK, CHUNK)

  dh3 = _sc_deg(dst3)
  dh = dh3.reshape(NW, NACC)[:, :N].T
  y0, root0 = _tc_stage_a(x, embeddings, Wr0, Wn0, Wp0, b0r)
  p0, p1 = _sc_pass0(y0, src, dst)
  y1, root1, recip = _tc_stage_b(p0, p1, dh, root0, embeddings,
                                 Wr1, Wn1, Wp1, b1r)
  q0, q1 = _sc_pass1(y1, src, dst)
  y2, root2 = _tc_stage_c(q0, q1, recip, root1, embeddings,
                          Wr2, Wn2, Wp2, b2r)
  s0, s1 = _sc_pass2(y2, src, dst)
  return _tc_stage_d(s0, s1, recip, root2)
